# baseline (speedup 1.0000x reference)
import functools

import jax
import jax.numpy as jnp
from jax import lax
from jax.experimental import pallas as pl
from jax.experimental.pallas import tpu as pltpu
from jax.experimental.pallas import tpu_sc as plsc

N = 10000
E = 160000
F_IN = 256
F_H = 128
H = 512
NPAD = 10240
CH = 125
NCH1 = E // 16 // CH
NCH2 = E // 32 // CH
RPT = NPAD // 16
A2W = 80
A2C = E // 16 // A2W
EPT = E // 16

_mesh = plsc.VectorSubcoreMesh(core_axis_name="c", subcore_axis_name="s")


def _deg_body(dst_hbm, out0_hbm, out1_hbm, idx_v, ones_v, zb_v, acc_sh):
    c = lax.axis_index("c")
    s = lax.axis_index("s")
    w = c * 16 + s
    for i in range(8):
        ones_v[pl.ds(i * 16, 16)] = jnp.full((16,), 1.0, jnp.float32)
    for i in range(RPT // 16):
        zb_v[pl.ds(i * 16, 16)] = jnp.zeros((16,), jnp.float32)
    pltpu.sync_copy(dst_hbm.at[w], idx_v)
    pltpu.sync_copy(zb_v, acc_sh.at[pl.ds(s * RPT, RPT)])
    plsc.subcore_barrier()

    def body(j, carry):
        pltpu.sync_copy(ones_v.at[pl.ds(0, CH)],
                        acc_sh.at[idx_v.at[j]], add=True)
        return carry

    lax.fori_loop(0, NCH2, body, 0)
    plsc.subcore_barrier()

    @pl.when(c == 0)
    def _():
        pltpu.sync_copy(acc_sh.at[pl.ds(s * RPT, RPT)],
                        out0_hbm.at[pl.ds(s * RPT, RPT)])

    @pl.when(c == 1)
    def _():
        pltpu.sync_copy(acc_sh.at[pl.ds(s * RPT, RPT)],
                        out1_hbm.at[pl.ds(s * RPT, RPT)])


_deg = functools.partial(
    pl.kernel,
    out_type=[jax.ShapeDtypeStruct((NPAD,), jnp.float32),
              jax.ShapeDtypeStruct((NPAD,), jnp.float32)],
    mesh=_mesh,
    scratch_types=[
        pltpu.VMEM((NCH2, CH), jnp.int32),
        pltpu.VMEM((128,), jnp.float32),
        pltpu.VMEM((RPT,), jnp.float32),
        pltpu.VMEM_SHARED((NPAD,), jnp.float32),
    ],
)(_deg_body)


def _prep_body(x_ref, p0_ref, p1_ref, xs_ref, d_ref):
    deg = p0_ref[...] + p1_ref[...] + 1.0
    dd1 = lax.rsqrt(deg)
    d_ref[...] = dd1
    dd = dd1[:, None]
    xb = x_ref[...] * dd
    xs_ref[0] = xb[:, :F_H]
    xs_ref[1] = xb[:, F_H:]


_BP = 2048

_prep = pl.pallas_call(
    _prep_body,
    grid=(NPAD // _BP,),
    in_specs=[
        pl.BlockSpec((_BP, F_IN), lambda m: (m, 0)),
        pl.BlockSpec((_BP,), lambda m: (m,)),
        pl.BlockSpec((_BP,), lambda m: (m,)),
    ],
    out_specs=[
        pl.BlockSpec((2, _BP, F_H), lambda m: (0, m, 0)),
        pl.BlockSpec((_BP,), lambda m: (m,)),
    ],
    out_shape=[
        jax.ShapeDtypeStruct((2, NPAD, F_H), jnp.float32),
        jax.ShapeDtypeStruct((NPAD,), jnp.float32),
    ],
)


_NB = 4


_HB = NCH1 // 2


def _agg1_body(xs2_hbm, srcs_hbm, dst_hbm, out_hbm,
               sidx_v, didx_v, buf_v, acc_sh, g0, g1, s0, s1):
    c = lax.axis_index("c")
    s = lax.axis_index("s")
    pltpu.sync_copy(xs2_hbm.at[pl.ds(c * NPAD + s * RPT, RPT)],
                    acc_sh.at[pl.ds(s * RPT, RPT)])
    plsc.subcore_barrier()

    table = xs2_hbm.at[pl.ds(c * NPAD, NPAD)]
    slot_a = buf_v.at[pl.ds(0, CH)]
    slot_b = buf_v.at[pl.ds(128, CH)]

    def _gather(j, slot, sem):
        pltpu.async_copy(table.at[sidx_v.at[j]], slot, sem)

    def _wait_gather(slot, sem):
        pltpu.make_async_copy(table.at[sidx_v.at[0]], slot, sem).wait()

    def _scatter(j, slot, sem):
        pltpu.async_copy(slot, acc_sh.at[didx_v.at[j]], sem, add=True)

    def _wait_scatter(slot, sem):
        pltpu.make_async_copy(slot, acc_sh.at[didx_v.at[0]], sem).wait()

    for h in range(2):
        pltpu.sync_copy(srcs_hbm.at[s, pl.ds(h * _HB, _HB)], sidx_v)
        pltpu.sync_copy(dst_hbm.at[s, pl.ds(h * _HB, _HB)], didx_v)
        _gather(0, slot_a, g0)
        _gather(1, slot_b, g1)

        def body(t, carry):
            j0 = t * 2
            _wait_gather(slot_a, g0)
            _scatter(j0, slot_a, s0)
            _wait_gather(slot_b, g1)
            _scatter(j0 + 1, slot_b, s1)
            _wait_scatter(slot_a, s0)

            @pl.when(j0 + 2 < _HB)
            def _():
                _gather(j0 + 2, slot_a, g0)

            _wait_scatter(slot_b, s1)

            @pl.when(j0 + 3 < _HB)
            def _():
                _gather(j0 + 3, slot_b, g1)

            return carry

        lax.fori_loop(0, _HB // 2, body, 0)
    plsc.subcore_barrier()
    pltpu.sync_copy(acc_sh.at[pl.ds(s * RPT, RPT)],
                    out_hbm.at[c, pl.ds(s * RPT, RPT)])


_agg1 = functools.partial(
    pl.kernel,
    out_type=jax.ShapeDtypeStruct((2, NPAD, F_H), jnp.float32),
    mesh=_mesh,
    scratch_types=[
        pltpu.VMEM((_HB, CH), jnp.int32),
        pltpu.VMEM((_HB, CH), jnp.int32),
        pltpu.VMEM((256, F_H), jnp.float32),
        pltpu.VMEM_SHARED((NPAD, F_H), jnp.float32),
        pltpu.SemaphoreType.DMA,
        pltpu.SemaphoreType.DMA,
        pltpu.SemaphoreType.DMA,
        pltpu.SemaphoreType.DMA,
    ],
)(_agg1_body)


_BM = 512


def _mm_body(a_ref, d_ref, w1_ref, b1_ref, w2_ref, zs_ref):
    db = d_ref[...][:, None]
    a = jnp.concatenate(
        [a_ref[0] * db, a_ref[1] * db], axis=1).astype(jnp.bfloat16)
    h = jnp.dot(a, w1_ref[...], preferred_element_type=jnp.float32)
    h = jnp.maximum(h + b1_ref[...], 0.0)
    z = jnp.dot(h.astype(jnp.bfloat16), w2_ref[...],
                preferred_element_type=jnp.float32)
    zs_ref[...] = d_ref[...] * z[:, 0]


_mm = pl.pallas_call(
    _mm_body,
    grid=(NPAD // _BM,),
    in_specs=[
        pl.BlockSpec((2, _BM, F_H), lambda m: (0, m, 0)),
        pl.BlockSpec((_BM,), lambda m: (m,)),
        pl.BlockSpec((F_IN, H), lambda m: (0, 0)),
        pl.BlockSpec((1, H), lambda m: (0, 0)),
        pl.BlockSpec((H, 1), lambda m: (0, 0)),
    ],
    out_specs=pl.BlockSpec((_BM,), lambda m: (m,)),
    out_shape=jax.ShapeDtypeStruct((NPAD,), jnp.float32),
)


_A2E = 10240
_A2NC = _A2E // 128


def _agg2_body(zs_hbm, srcs_hbm, dst_hbm, d_hbm, b2_hbm, out_hbm,
               zsb_v, sidx_v, didx_v, vals_v, db_v, b2_v, accb_v, acc_sh,
               s0, s1):
    c = lax.axis_index("c")
    s = lax.axis_index("s")

    @pl.when(c == 0)
    def _():
        pltpu.sync_copy(zs_hbm, zsb_v)
        pltpu.sync_copy(srcs_hbm.at[pl.ds(s * _A2E, _A2E)], sidx_v)
        pltpu.sync_copy(dst_hbm.at[s], didx_v)
        pltpu.sync_copy(d_hbm.at[pl.ds(s * RPT, RPT)], db_v)
        pltpu.sync_copy(b2_hbm, b2_v)
        pltpu.sync_copy(zs_hbm.at[pl.ds(s * RPT, RPT)],
                        acc_sh.at[pl.ds(s * RPT, RPT)])
        plsc.subcore_barrier()

        def gbody(j, carry):
            vidx = sidx_v[pl.ds(j * 16, 16)]
            vals_v[pl.ds(j * 16, 16)] = plsc.load_gather(zsb_v, [vidx])
            return carry

        lax.fori_loop(0, _A2E // 16, gbody, 0)

        def sbody(t, carry):
            j0 = t * 2
            sa = pltpu.async_copy(vals_v.at[pl.ds(j0 * 128, 128)],
                                  acc_sh.at[didx_v.at[j0]], s0, add=True)
            sb = pltpu.async_copy(vals_v.at[pl.ds(j0 * 128 + 128, 128)],
                                  acc_sh.at[didx_v.at[j0 + 1]], s1, add=True)
            sa.wait()
            sb.wait()
            return carry

        lax.fori_loop(0, _A2NC // 2, sbody, 0)
        plsc.subcore_barrier()
        pltpu.sync_copy(acc_sh.at[pl.ds(s * RPT, RPT)], accb_v)
        b2v = b2_v[pl.ds(0, 16)]
        for k in range(RPT // 16):
            sl = pl.ds(k * 16, 16)
            accb_v[sl] = db_v[sl] * accb_v[sl] + b2v
        pltpu.sync_copy(accb_v, out_hbm.at[pl.ds(s * RPT, RPT)])


_agg2 = functools.partial(
    pl.kernel,
    out_type=jax.ShapeDtypeStruct((NPAD,), jnp.float32),
    mesh=_mesh,
    compiler_params=pltpu.CompilerParams(needs_layout_passes=False),
    scratch_types=[
        pltpu.VMEM((NPAD,), jnp.float32),
        pltpu.VMEM((_A2E,), jnp.int32),
        pltpu.VMEM((_A2NC, 128), jnp.int32),
        pltpu.VMEM((_A2E,), jnp.float32),
        pltpu.VMEM((RPT,), jnp.float32),
        pltpu.VMEM((16,), jnp.float32),
        pltpu.VMEM((RPT,), jnp.float32),
        pltpu.VMEM_SHARED((NPAD,), jnp.float32),
        pltpu.SemaphoreType.DMA,
        pltpu.SemaphoreType.DMA,
    ],
)(_agg2_body)


def kernel(x, edge_index, W1, b1, W2, b2):
    x = x.astype(jnp.float32)
    ei = edge_index.astype(jnp.int32)
    src = ei[0]
    dst = ei[1]
    dst32 = dst.reshape(32, NCH2, CH)
    srcs16 = src.reshape(16, NCH1, CH)
    dst16 = dst.reshape(16, NCH1, CH)
    padn = 16 * _A2E - E
    pidx = jnp.arange(padn, dtype=jnp.int32)
    srcs_a2 = jnp.concatenate([src, (pidx * 97) % N])
    dst_a2 = jnp.concatenate([dst, N + (pidx % (NPAD - N))]).reshape(
        16, _A2NC, 128)

    p0, p1 = _deg(dst32)
    xs, d = _prep(x, p0, p1)
    xs2 = xs.reshape(2 * NPAD, F_H)
    acc = _agg1(xs2, srcs16, dst16)
    zs = _mm(acc, d, W1.astype(jnp.bfloat16), b1.reshape(1, H),
             W2.astype(jnp.bfloat16))
    out = _agg2(zs, srcs_a2, dst_a2, d,
                jnp.broadcast_to(b2.astype(jnp.float32), (16,)))
    return out[:N]


# speedup vs baseline: 1.0265x; 1.0265x over previous
import functools

import jax
import jax.numpy as jnp
from jax import lax
from jax.experimental import pallas as pl
from jax.experimental.pallas import tpu as pltpu
from jax.experimental.pallas import tpu_sc as plsc

N = 10000
E = 160000
F_IN = 256
F_H = 128
H = 512
NPAD = 10240
CH = 125
NCH1 = E // 16 // CH
NCH2 = E // 32 // CH
RPT = NPAD // 16
A2W = 80
A2C = E // 16 // A2W
EPT = E // 16

_mesh = plsc.VectorSubcoreMesh(core_axis_name="c", subcore_axis_name="s")


def _deg_body(dst_hbm, out0_hbm, out1_hbm, idx_v, ones_v, zb_v, acc_sh):
    c = lax.axis_index("c")
    s = lax.axis_index("s")
    w = c * 16 + s
    for i in range(8):
        ones_v[pl.ds(i * 16, 16)] = jnp.full((16,), 1.0, jnp.float32)
    for i in range(RPT // 16):
        zb_v[pl.ds(i * 16, 16)] = jnp.zeros((16,), jnp.float32)
    pltpu.sync_copy(dst_hbm.at[w], idx_v)
    pltpu.sync_copy(zb_v, acc_sh.at[pl.ds(s * RPT, RPT)])
    plsc.subcore_barrier()

    def body(j, carry):
        pltpu.sync_copy(ones_v.at[pl.ds(0, CH)],
                        acc_sh.at[idx_v.at[j]], add=True)
        return carry

    lax.fori_loop(0, NCH2, body, 0)
    plsc.subcore_barrier()

    @pl.when(c == 0)
    def _():
        pltpu.sync_copy(acc_sh.at[pl.ds(s * RPT, RPT)],
                        out0_hbm.at[pl.ds(s * RPT, RPT)])

    @pl.when(c == 1)
    def _():
        pltpu.sync_copy(acc_sh.at[pl.ds(s * RPT, RPT)],
                        out1_hbm.at[pl.ds(s * RPT, RPT)])


_deg = functools.partial(
    pl.kernel,
    out_type=[jax.ShapeDtypeStruct((NPAD,), jnp.float32),
              jax.ShapeDtypeStruct((NPAD,), jnp.float32)],
    mesh=_mesh,
    scratch_types=[
        pltpu.VMEM((NCH2, CH), jnp.int32),
        pltpu.VMEM((128,), jnp.float32),
        pltpu.VMEM((RPT,), jnp.float32),
        pltpu.VMEM_SHARED((NPAD,), jnp.float32),
    ],
)(_deg_body)


def _prep_body(x_ref, p0_ref, p1_ref, xs_ref, d_ref):
    deg = p0_ref[...] + p1_ref[...] + 1.0
    dd1 = lax.rsqrt(deg)
    d_ref[...] = dd1
    dd = dd1[:, None]
    xb = x_ref[...] * dd
    xs_ref[0] = xb[:, :F_H]
    xs_ref[1] = xb[:, F_H:]


_BP = 2048

_prep = pl.pallas_call(
    _prep_body,
    grid=(NPAD // _BP,),
    in_specs=[
        pl.BlockSpec((_BP, F_IN), lambda m: (m, 0)),
        pl.BlockSpec((_BP,), lambda m: (m,)),
        pl.BlockSpec((_BP,), lambda m: (m,)),
    ],
    out_specs=[
        pl.BlockSpec((2, _BP, F_H), lambda m: (0, m, 0)),
        pl.BlockSpec((_BP,), lambda m: (m,)),
    ],
    out_shape=[
        jax.ShapeDtypeStruct((2, NPAD, F_H), jnp.float32),
        jax.ShapeDtypeStruct((NPAD,), jnp.float32),
    ],
)


_NB = 4


_HB = NCH1 // 2


def _agg1_body(xs2_hbm, srcs_hbm, dst_hbm, out_hbm,
               sidx_v, didx_v, buf_v, acc_sh, g0, g1, s0, s1):
    c = lax.axis_index("c")
    s = lax.axis_index("s")
    pltpu.sync_copy(xs2_hbm.at[pl.ds(c * NPAD + s * RPT, RPT)],
                    acc_sh.at[pl.ds(s * RPT, RPT)])
    plsc.subcore_barrier()

    table = xs2_hbm.at[pl.ds(c * NPAD, NPAD)]
    slot_a = buf_v.at[pl.ds(0, CH)]
    slot_b = buf_v.at[pl.ds(128, CH)]

    def _gather(j, slot, sem):
        pltpu.async_copy(table.at[sidx_v.at[j]], slot, sem)

    def _wait_gather(slot, sem):
        pltpu.make_async_copy(table.at[sidx_v.at[0]], slot, sem).wait()

    def _scatter(j, slot, sem):
        pltpu.async_copy(slot, acc_sh.at[didx_v.at[j]], sem, add=True)

    def _wait_scatter(slot, sem):
        pltpu.make_async_copy(slot, acc_sh.at[didx_v.at[0]], sem).wait()

    for h in range(2):
        pltpu.sync_copy(srcs_hbm.at[s, pl.ds(h * _HB, _HB)], sidx_v)
        pltpu.sync_copy(dst_hbm.at[s, pl.ds(h * _HB, _HB)], didx_v)
        _gather(0, slot_a, g0)
        _gather(1, slot_b, g1)

        def body(t, carry):
            j0 = t * 2
            _wait_gather(slot_a, g0)
            _scatter(j0, slot_a, s0)
            _wait_gather(slot_b, g1)
            _scatter(j0 + 1, slot_b, s1)
            _wait_scatter(slot_a, s0)

            @pl.when(j0 + 2 < _HB)
            def _():
                _gather(j0 + 2, slot_a, g0)

            _wait_scatter(slot_b, s1)

            @pl.when(j0 + 3 < _HB)
            def _():
                _gather(j0 + 3, slot_b, g1)

            return carry

        lax.fori_loop(0, _HB // 2, body, 0)
    plsc.subcore_barrier()
    pltpu.sync_copy(acc_sh.at[pl.ds(s * RPT, RPT)],
                    out_hbm.at[c, pl.ds(s * RPT, RPT)])


_agg1 = functools.partial(
    pl.kernel,
    out_type=jax.ShapeDtypeStruct((2, NPAD, F_H), jnp.float32),
    mesh=_mesh,
    scratch_types=[
        pltpu.VMEM((_HB, CH), jnp.int32),
        pltpu.VMEM((_HB, CH), jnp.int32),
        pltpu.VMEM((256, F_H), jnp.float32),
        pltpu.VMEM_SHARED((NPAD, F_H), jnp.float32),
        pltpu.SemaphoreType.DMA,
        pltpu.SemaphoreType.DMA,
        pltpu.SemaphoreType.DMA,
        pltpu.SemaphoreType.DMA,
    ],
)(_agg1_body)


_BM = 1024


def _mm_body(a_ref, d_ref, w1_ref, b1_ref, w2_ref, zs_ref):
    db = d_ref[...][:, None]
    a = jnp.concatenate(
        [a_ref[0] * db, a_ref[1] * db], axis=1).astype(jnp.bfloat16)
    h = jnp.dot(a, w1_ref[...], preferred_element_type=jnp.float32)
    h = jnp.maximum(h + b1_ref[...], 0.0)
    z = jnp.dot(h.astype(jnp.bfloat16), w2_ref[...],
                preferred_element_type=jnp.float32)
    zs_ref[...] = d_ref[...] * z[:, 0]


_mm = pl.pallas_call(
    _mm_body,
    grid=(NPAD // _BM,),
    in_specs=[
        pl.BlockSpec((2, _BM, F_H), lambda m: (0, m, 0)),
        pl.BlockSpec((_BM,), lambda m: (m,)),
        pl.BlockSpec((F_IN, H), lambda m: (0, 0)),
        pl.BlockSpec((1, H), lambda m: (0, 0)),
        pl.BlockSpec((H, 1), lambda m: (0, 0)),
    ],
    out_specs=pl.BlockSpec((_BM,), lambda m: (m,)),
    out_shape=jax.ShapeDtypeStruct((NPAD,), jnp.float32),
)


_A2E = 10240
_A2NC = _A2E // 128


def _agg2_body(zs_hbm, srcs_hbm, dst_hbm, d_hbm, b2_hbm, out_hbm,
               zsb_v, sidx_v, didx_v, vals_v, db_v, b2_v, accb_v, acc_sh,
               s0, s1):
    c = lax.axis_index("c")
    s = lax.axis_index("s")

    @pl.when(c == 0)
    def _():
        pltpu.sync_copy(zs_hbm, zsb_v)
        pltpu.sync_copy(srcs_hbm.at[pl.ds(s * _A2E, _A2E)], sidx_v)
        pltpu.sync_copy(dst_hbm.at[s], didx_v)
        pltpu.sync_copy(d_hbm.at[pl.ds(s * RPT, RPT)], db_v)
        pltpu.sync_copy(b2_hbm, b2_v)
        pltpu.sync_copy(zs_hbm.at[pl.ds(s * RPT, RPT)],
                        acc_sh.at[pl.ds(s * RPT, RPT)])
        plsc.subcore_barrier()

        def gbody(j, carry):
            vidx = sidx_v[pl.ds(j * 16, 16)]
            vals_v[pl.ds(j * 16, 16)] = plsc.load_gather(zsb_v, [vidx])
            return carry

        lax.fori_loop(0, _A2E // 16, gbody, 0)

        def sbody(t, carry):
            j0 = t * 2
            sa = pltpu.async_copy(vals_v.at[pl.ds(j0 * 128, 128)],
                                  acc_sh.at[didx_v.at[j0]], s0, add=True)
            sb = pltpu.async_copy(vals_v.at[pl.ds(j0 * 128 + 128, 128)],
                                  acc_sh.at[didx_v.at[j0 + 1]], s1, add=True)
            sa.wait()
            sb.wait()
            return carry

        lax.fori_loop(0, _A2NC // 2, sbody, 0)
        plsc.subcore_barrier()
        pltpu.sync_copy(acc_sh.at[pl.ds(s * RPT, RPT)], accb_v)
        b2v = b2_v[pl.ds(0, 16)]
        for k in range(RPT // 16):
            sl = pl.ds(k * 16, 16)
            accb_v[sl] = db_v[sl] * accb_v[sl] + b2v
        pltpu.sync_copy(accb_v, out_hbm.at[pl.ds(s * RPT, RPT)])


_agg2 = functools.partial(
    pl.kernel,
    out_type=jax.ShapeDtypeStruct((NPAD,), jnp.float32),
    mesh=_mesh,
    compiler_params=pltpu.CompilerParams(needs_layout_passes=False),
    scratch_types=[
        pltpu.VMEM((NPAD,), jnp.float32),
        pltpu.VMEM((_A2E,), jnp.int32),
        pltpu.VMEM((_A2NC, 128), jnp.int32),
        pltpu.VMEM((_A2E,), jnp.float32),
        pltpu.VMEM((RPT,), jnp.float32),
        pltpu.VMEM((16,), jnp.float32),
        pltpu.VMEM((RPT,), jnp.float32),
        pltpu.VMEM_SHARED((NPAD,), jnp.float32),
        pltpu.SemaphoreType.DMA,
        pltpu.SemaphoreType.DMA,
    ],
)(_agg2_body)


def kernel(x, edge_index, W1, b1, W2, b2):
    x = x.astype(jnp.float32)
    ei = edge_index.astype(jnp.int32)
    src = ei[0]
    dst = ei[1]
    dst32 = dst.reshape(32, NCH2, CH)
    srcs16 = src.reshape(16, NCH1, CH)
    dst16 = dst.reshape(16, NCH1, CH)
    padn = 16 * _A2E - E
    pidx = jnp.arange(padn, dtype=jnp.int32)
    srcs_a2 = jnp.concatenate([src, (pidx * 97) % N])
    dst_a2 = jnp.concatenate([dst, N + (pidx % (NPAD - N))]).reshape(
        16, _A2NC, 128)

    p0, p1 = _deg(dst32)
    xs, d = _prep(x, p0, p1)
    xs2 = xs.reshape(2 * NPAD, F_H)
    acc = _agg1(xs2, srcs16, dst16)
    zs = _mm(acc, d, W1.astype(jnp.bfloat16), b1.reshape(1, H),
             W2.astype(jnp.bfloat16))
    out = _agg2(zs, srcs_a2, dst_a2, d,
                jnp.broadcast_to(b2.astype(jnp.float32), (16,)))
    return out[:N]
